# baseline (speedup 1.0000x reference)
import jax
import jax.numpy as jnp
from jax import lax
from jax.experimental import pallas as pl
from jax.experimental.pallas import tpu as pltpu
from jax.experimental.pallas import tpu_sc as plsc

_B, _S, _F = 2, 8192, 2048
_CH = 256
_NW = 32
_CW = 128
_NCOL = _F // _CW
_JOBS = _B * _NCOL // _NW
_NG = _CW // 16


def _sc_body(x_hbm, o_hbm, buf, obuf):
    wid = lax.axis_index("s") * 2 + lax.axis_index("c")

    def job_body(j, carry):
        job = wid * _JOBS + j
        b = job // _NCOL
        f0 = (job % _NCOL) * _CW

        def chunk_body(k, accs):
            s0 = k * _CH
            pltpu.sync_copy(x_hbm.at[b, pl.ds(s0, _CH), pl.ds(f0, _CW)], buf)

            def row(i, accs):
                new = []
                for g in range(_NG):
                    a = accs[g] + buf[i, g * 16:(g + 1) * 16]
                    obuf[i, g * 16:(g + 1) * 16] = a
                    new.append(a)
                return tuple(new)

            accs = lax.fori_loop(0, _CH, row, accs)
            pltpu.sync_copy(obuf, o_hbm.at[b, pl.ds(s0, _CH), pl.ds(f0, _CW)])
            return accs

        zero = tuple(jnp.zeros((16,), jnp.float32) for _ in range(_NG))
        lax.fori_loop(0, _S // _CH, chunk_body, zero)
        return carry

    lax.fori_loop(0, _JOBS, job_body, 0)


def kernel(x, dim):
    mesh = plsc.VectorSubcoreMesh(core_axis_name="c", subcore_axis_name="s")
    f = pl.kernel(
        _sc_body,
        out_type=jax.ShapeDtypeStruct((_B, _S, _F), jnp.float32),
        mesh=mesh,
        scratch_types=[
            pltpu.VMEM((_CH, _CW), jnp.float32),
            pltpu.VMEM((_CH, _CW), jnp.float32),
        ],
    )
    return f(x)


# speedup vs baseline: 1.4516x; 1.4516x over previous
import jax
import jax.numpy as jnp
from jax import lax
from jax.experimental import pallas as pl
from jax.experimental.pallas import tpu as pltpu
from jax.experimental.pallas import tpu_sc as plsc

_B, _S, _F = 2, 8192, 2048
_CH = 128
_CW = 128
_NCOL = _F // _CW
_NG = _CW // 16
_NCHUNKS = _S // _CH


def _sc_body(x_hbm, o_hbm, bi0, bi1, bo0, bo1, si0, si1, so0, so1):
    wid = lax.axis_index("s") * 2 + lax.axis_index("c")
    b = wid // _NCOL
    f0 = (wid % _NCOL) * _CW

    ibufs, obufs = (bi0, bi1), (bo0, bo1)
    isems, osems = (si0, si1), (so0, so1)

    def dma_in(k, slot):
        return pltpu.async_copy(
            x_hbm.at[b, pl.ds(k * _CH, _CH), pl.ds(f0, _CW)],
            ibufs[slot], isems[slot])

    def dma_out(k, slot):
        return pltpu.async_copy(
            obufs[slot], o_hbm.at[b, pl.ds(k * _CH, _CH), pl.ds(f0, _CW)],
            osems[slot])

    accs = tuple(jnp.zeros((16,), jnp.float32) for _ in range(_NG))
    h_in = [dma_in(0, 0), None]
    h_out = [None, None]
    for k in range(_NCHUNKS):
        slot = k & 1
        if k + 1 < _NCHUNKS:
            h_in[1 - slot] = dma_in(k + 1, 1 - slot)
        h_in[slot].wait()
        if h_out[slot] is not None:
            h_out[slot].wait()
        buf, obuf = ibufs[slot], obufs[slot]

        def row(i, accs):
            new = []
            for g in range(_NG):
                a = accs[g] + buf[i, g * 16:(g + 1) * 16]
                obuf[i, g * 16:(g + 1) * 16] = a
                new.append(a)
            return tuple(new)

        accs = lax.fori_loop(0, _CH, row, accs)
        h_out[slot] = dma_out(k, slot)
    h_out[0].wait()
    h_out[1].wait()


def kernel(x, dim):
    mesh = plsc.VectorSubcoreMesh(core_axis_name="c", subcore_axis_name="s")
    f = pl.kernel(
        _sc_body,
        out_type=jax.ShapeDtypeStruct((_B, _S, _F), jnp.float32),
        mesh=mesh,
        scratch_types=[
            pltpu.VMEM((_CH, _CW), jnp.float32),
            pltpu.VMEM((_CH, _CW), jnp.float32),
            pltpu.VMEM((_CH, _CW), jnp.float32),
            pltpu.VMEM((_CH, _CW), jnp.float32),
            pltpu.SemaphoreType.DMA,
            pltpu.SemaphoreType.DMA,
            pltpu.SemaphoreType.DMA,
            pltpu.SemaphoreType.DMA,
        ],
    )
    return f(x)
